# baseline (speedup 1.0000x reference)
import functools

import jax
import jax.numpy as jnp
from jax.experimental import pallas as pl
from jax.experimental.pallas import tpu as pltpu

N_TOK_BLK = 512
N_CODE_BLK = 2048


def _vq_body(ze_ref, w_ref, zen_ref, wn_ref,
             zq_ref, idx_ref, commit_ref,
             racc_ref, ridx_ref, rrow_ref, cacc_ref,
             *, n_tok_blocks, n_code_blocks, inv_count):
    i = pl.program_id(0)
    j = pl.program_id(1)
    tb = ze_ref.shape[0]
    cb = w_ref.shape[0]

    ze = ze_ref[...]
    w = w_ref[...]
    mm = jax.lax.dot_general(ze, w, (((1,), (1,)), ((), ())),
                             preferred_element_type=jnp.float32)
    dist = (zen_ref[...] - 2.0 * mm) + wn_ref[...]

    bmin = jnp.min(dist, axis=1, keepdims=True)
    col = jax.lax.broadcasted_iota(jnp.int32, (tb, cb), 1)
    bidx = jnp.min(jnp.where(dist == bmin, col, 2 ** 30),
                   axis=1, keepdims=True)
    onehot = (col == bidx).astype(jnp.float32)
    cand = jax.lax.dot_general(onehot, w, (((1,), (0,)), ((), ())),
                               preferred_element_type=jnp.float32,
                               precision=jax.lax.Precision.HIGHEST)
    bmin_bf = bmin.astype(jnp.bfloat16).astype(jnp.float32)

    @pl.when(j == 0)
    def _():
        racc_ref[...] = bmin_bf
        ridx_ref[...] = bidx
        rrow_ref[...] = cand

    @pl.when(j > 0)
    def _():
        upd = bmin < racc_ref[...]
        racc_ref[...] = jnp.where(upd, bmin_bf, racc_ref[...])
        ridx_ref[...] = jnp.where(upd, bidx + j * cb, ridx_ref[...])
        rrow_ref[...] = jnp.where(upd, cand, rrow_ref[...])

    @pl.when(j == n_code_blocks - 1)
    def _():
        zq = rrow_ref[...]
        zq_ref[...] = ze + (zq - ze)
        idx_ref[...] = jnp.reshape(ridx_ref[...], (1, 1, tb))
        partial = jnp.reshape(jnp.sum((zq - ze) ** 2), (1, 1))
        first = (i == 0)

        @pl.when(first)
        def _():
            cacc_ref[...] = partial

        @pl.when(jnp.logical_not(first))
        def _():
            cacc_ref[...] = cacc_ref[...] + partial

        @pl.when(i == n_tok_blocks - 1)
        def _():
            commit_ref[...] = cacc_ref[...] * inv_count


def kernel(ze, embedW):
    B, T, D = ze.shape
    K = embedW.shape[0]
    N = B * T
    ze_flat = ze.reshape(-1, D)
    zen = jnp.sum(ze_flat ** 2, axis=1, keepdims=True)
    wn = jnp.sum(embedW.T ** 2, axis=0, keepdims=True)

    nt = N // N_TOK_BLK
    nk = K // N_CODE_BLK

    body = functools.partial(_vq_body, n_tok_blocks=nt, n_code_blocks=nk,
                             inv_count=1.0 / float(N * D))

    zq, idx3, commit = pl.pallas_call(
        body,
        grid=(nt, nk),
        in_specs=[
            pl.BlockSpec((N_TOK_BLK, D), lambda i, j: (i, 0)),
            pl.BlockSpec((N_CODE_BLK, D), lambda i, j: (j, 0)),
            pl.BlockSpec((N_TOK_BLK, 1), lambda i, j: (i, 0)),
            pl.BlockSpec((1, N_CODE_BLK), lambda i, j: (0, j)),
        ],
        out_specs=[
            pl.BlockSpec((N_TOK_BLK, D), lambda i, j: (i, 0)),
            pl.BlockSpec((1, 1, N_TOK_BLK), lambda i, j: (i, 0, 0)),
            pl.BlockSpec((1, 1), lambda i, j: (0, 0)),
        ],
        out_shape=[
            jax.ShapeDtypeStruct((N, D), jnp.float32),
            jax.ShapeDtypeStruct((nt, 1, N_TOK_BLK), jnp.int32),
            jax.ShapeDtypeStruct((1, 1), jnp.float32),
        ],
        scratch_shapes=[
            pltpu.VMEM((N_TOK_BLK, 1), jnp.float32),
            pltpu.VMEM((N_TOK_BLK, 1), jnp.int32),
            pltpu.VMEM((N_TOK_BLK, D), jnp.float32),
            pltpu.VMEM((1, 1), jnp.float32),
        ],
    )(ze_flat, embedW, zen, wn)

    return (zq.reshape(B, T, D), commit[0, 0], idx3.reshape(B, T))


# speedup vs baseline: 2.3910x; 2.3910x over previous
import functools

import jax
import jax.numpy as jnp
from jax import lax
from jax.experimental import pallas as pl
from jax.experimental.pallas import tpu as pltpu
from jax.experimental.pallas import tpu_sc as plsc

N_TOK_BLK = 512
N_CODE_BLK = 2048


def _argmin_body(ze2_ref, w_ref, zen_ref, wn_ref, idx_ref,
                 racc_ref, ridx_ref, *, n_code_blocks):
    j = pl.program_id(1)
    tb = ze2_ref.shape[0]
    cb = w_ref.shape[0]

    mm2 = jax.lax.dot_general(ze2_ref[...], w_ref[...], (((1,), (1,)), ((), ())),
                              preferred_element_type=jnp.float32)
    dist = (zen_ref[...] + mm2) + wn_ref[...]

    bmin = jnp.min(dist, axis=1, keepdims=True)
    col = jax.lax.broadcasted_iota(jnp.int32, (tb, cb), 1)
    bidx = jnp.min(jnp.where(dist == bmin, col, 2 ** 30),
                   axis=1, keepdims=True)
    bmin_bf = bmin.astype(jnp.bfloat16).astype(jnp.float32)

    @pl.when(j == 0)
    def _():
        racc_ref[...] = bmin_bf
        ridx_ref[...] = bidx

    @pl.when(j > 0)
    def _():
        upd = bmin < racc_ref[...]
        racc_ref[...] = jnp.where(upd, bmin_bf, racc_ref[...])
        ridx_ref[...] = jnp.where(upd, bidx + j * cb, ridx_ref[...])

    @pl.when(j == n_code_blocks - 1)
    def _():
        idx_ref[...] = jnp.reshape(ridx_ref[...], (1, 1, tb))


def _epilogue_body(ze_ref, zq_ref, st_ref, commit_ref, cacc_ref,
                   *, n_tok_blocks, inv_count):
    i = pl.program_id(0)
    ze = ze_ref[...]
    zq = zq_ref[:, : ze.shape[1]]
    st_ref[...] = ze + (zq - ze)
    partial = jnp.reshape(jnp.sum((zq - ze) ** 2), (1, 1))

    @pl.when(i == 0)
    def _():
        cacc_ref[...] = partial

    @pl.when(i > 0)
    def _():
        cacc_ref[...] = cacc_ref[...] + partial

    @pl.when(i == n_tok_blocks - 1)
    def _():
        commit_ref[...] = cacc_ref[...] * inv_count


def _sc_gather(table, idx, n, d):
    info = plsc.get_sparse_core_info()
    n_workers = info.num_cores * info.num_subcores
    b_per_w = n // n_workers
    chunk = 128
    n_chunks = b_per_w // chunk
    mesh = plsc.VectorSubcoreMesh(core_axis_name="c", subcore_axis_name="s")

    @functools.partial(
        pl.kernel, mesh=mesh,
        out_type=jax.ShapeDtypeStruct((n, d), jnp.float32),
        scratch_types=[
            pltpu.VMEM((chunk,), jnp.int32),
            pltpu.VMEM((b_per_w, d), jnp.float32),
            pltpu.SemaphoreType.DMA,
        ],
    )
    def gather_kernel(table_hbm, idx_hbm, out_hbm, idx_v, rows_v, sem):
        wid = lax.axis_index("s") * info.num_cores + lax.axis_index("c")
        base = wid * b_per_w
        for k in range(n_chunks):
            pltpu.sync_copy(idx_hbm.at[pl.ds(base + k * chunk, chunk)], idx_v)
            pltpu.async_copy(table_hbm.at[idx_v],
                             rows_v.at[pl.ds(k * chunk, chunk)], sem).wait()
        pltpu.sync_copy(rows_v, out_hbm.at[pl.ds(base, b_per_w)])

    return gather_kernel(table, idx)


def kernel(ze, embedW):
    B, T, D = ze.shape
    K = embedW.shape[0]
    N = B * T
    ze_flat = ze.reshape(-1, D)
    zen = jnp.sum(ze_flat ** 2, axis=1, keepdims=True)
    wn = jnp.sum(embedW.T ** 2, axis=0, keepdims=True)
    ze2 = -2.0 * ze_flat

    nt = N // N_TOK_BLK
    nk = K // N_CODE_BLK

    idx3 = pl.pallas_call(
        functools.partial(_argmin_body, n_code_blocks=nk),
        grid=(nt, nk),
        in_specs=[
            pl.BlockSpec((N_TOK_BLK, D), lambda i, j: (i, 0)),
            pl.BlockSpec((N_CODE_BLK, D), lambda i, j: (j, 0)),
            pl.BlockSpec((N_TOK_BLK, 1), lambda i, j: (i, 0)),
            pl.BlockSpec((1, N_CODE_BLK), lambda i, j: (0, j)),
        ],
        out_specs=pl.BlockSpec((1, 1, N_TOK_BLK), lambda i, j: (i, 0, 0)),
        out_shape=jax.ShapeDtypeStruct((nt, 1, N_TOK_BLK), jnp.int32),
        scratch_shapes=[
            pltpu.VMEM((N_TOK_BLK, 1), jnp.float32),
            pltpu.VMEM((N_TOK_BLK, 1), jnp.int32),
        ],
    )(ze2, embedW, zen, wn)

    idx_flat = idx3.reshape(N)
    table = jnp.pad(embedW, ((0, 0), (0, 128 - D)))
    zq_pad = _sc_gather(table, idx_flat, N, 128)

    zq_st, commit = pl.pallas_call(
        functools.partial(_epilogue_body, n_tok_blocks=nt,
                          inv_count=1.0 / float(N * D)),
        grid=(nt,),
        in_specs=[
            pl.BlockSpec((N_TOK_BLK, D), lambda i: (i, 0)),
            pl.BlockSpec((N_TOK_BLK, 128), lambda i: (i, 0)),
        ],
        out_specs=[
            pl.BlockSpec((N_TOK_BLK, D), lambda i: (i, 0)),
            pl.BlockSpec((1, 1), lambda i: (0, 0)),
        ],
        out_shape=[
            jax.ShapeDtypeStruct((N, D), jnp.float32),
            jax.ShapeDtypeStruct((1, 1), jnp.float32),
        ],
        scratch_shapes=[pltpu.VMEM((1, 1), jnp.float32)],
    )(ze_flat, zq_pad)

    return (zq_st.reshape(B, T, D), commit[0, 0], idx3.reshape(B, T))


# speedup vs baseline: 2.5968x; 1.0861x over previous
import functools

import jax
import jax.numpy as jnp
from jax import lax
from jax.experimental import pallas as pl
from jax.experimental.pallas import tpu as pltpu
from jax.experimental.pallas import tpu_sc as plsc

N_TOK_BLK = 1024
N_CODE_BLK = 2048


def _argmin_body(ze2_ref, w_ref, zen_ref, wn_ref, idx_ref,
                 racc_ref, ridx_ref, *, n_code_blocks):
    j = pl.program_id(1)
    tb = ze2_ref.shape[0]
    cb = w_ref.shape[0]

    mm2 = jax.lax.dot_general(ze2_ref[...], w_ref[...], (((1,), (1,)), ((), ())),
                              preferred_element_type=jnp.float32)
    dist = (zen_ref[...] + mm2) + wn_ref[...]

    bmin = jnp.min(dist, axis=1, keepdims=True)
    col = jax.lax.broadcasted_iota(jnp.int32, (tb, cb), 1)
    bidx = jnp.min(jnp.where(dist == bmin, col, 2 ** 30),
                   axis=1, keepdims=True)
    bmin_bf = bmin.astype(jnp.bfloat16).astype(jnp.float32)

    @pl.when(j == 0)
    def _():
        racc_ref[...] = bmin_bf
        ridx_ref[...] = bidx

    @pl.when(j > 0)
    def _():
        upd = bmin < racc_ref[...]
        racc_ref[...] = jnp.where(upd, bmin_bf, racc_ref[...])
        ridx_ref[...] = jnp.where(upd, bidx + j * cb, ridx_ref[...])

    @pl.when(j == n_code_blocks - 1)
    def _():
        idx_ref[...] = jnp.reshape(ridx_ref[...], (1, 1, tb))


def _epilogue_body(ze_ref, zq_ref, st_ref, commit_ref, cacc_ref,
                   *, n_tok_blocks, inv_count):
    i = pl.program_id(0)
    ze = ze_ref[...]
    zq = zq_ref[:, : ze.shape[1]]
    st_ref[...] = ze + (zq - ze)
    partial = jnp.reshape(jnp.sum((zq - ze) ** 2), (1, 1))

    @pl.when(i == 0)
    def _():
        cacc_ref[...] = partial

    @pl.when(i > 0)
    def _():
        cacc_ref[...] = cacc_ref[...] + partial

    @pl.when(i == n_tok_blocks - 1)
    def _():
        commit_ref[...] = cacc_ref[...] * inv_count


def _sc_gather(table, idx, n, d):
    info = plsc.get_sparse_core_info()
    n_workers = info.num_cores * info.num_subcores
    b_per_w = n // n_workers
    chunk = 128
    n_chunks = b_per_w // chunk
    mesh = plsc.VectorSubcoreMesh(core_axis_name="c", subcore_axis_name="s")

    @functools.partial(
        pl.kernel, mesh=mesh,
        out_type=jax.ShapeDtypeStruct((n, d), jnp.float32),
        scratch_types=[
            pltpu.VMEM((n_chunks, chunk), jnp.int32),
            pltpu.VMEM((b_per_w, d), jnp.float32),
            pltpu.SemaphoreType.DMA,
        ],
    )
    def gather_kernel(table_hbm, idx_hbm, out_hbm, idx_v, rows_v, sem):
        wid = lax.axis_index("s") * info.num_cores + lax.axis_index("c")
        base = wid * b_per_w
        copies = []
        for k in range(n_chunks):
            pltpu.sync_copy(idx_hbm.at[pl.ds(base + k * chunk, chunk)],
                            idx_v.at[k])
            copies.append(pltpu.async_copy(
                table_hbm.at[idx_v.at[k]],
                rows_v.at[pl.ds(k * chunk, chunk)], sem))
        for c in copies:
            c.wait()
        pltpu.sync_copy(rows_v, out_hbm.at[pl.ds(base, b_per_w)])

    return gather_kernel(table, idx)


def kernel(ze, embedW):
    B, T, D = ze.shape
    K = embedW.shape[0]
    N = B * T
    ze_flat = ze.reshape(-1, D)
    zen = jnp.sum(ze_flat ** 2, axis=1, keepdims=True)
    wn = jnp.sum(embedW.T ** 2, axis=0, keepdims=True)
    ze2 = -2.0 * ze_flat

    nt = N // N_TOK_BLK
    nk = K // N_CODE_BLK

    idx3 = pl.pallas_call(
        functools.partial(_argmin_body, n_code_blocks=nk),
        grid=(nt, nk),
        in_specs=[
            pl.BlockSpec((N_TOK_BLK, D), lambda i, j: (i, 0)),
            pl.BlockSpec((N_CODE_BLK, D), lambda i, j: (j, 0)),
            pl.BlockSpec((N_TOK_BLK, 1), lambda i, j: (i, 0)),
            pl.BlockSpec((1, N_CODE_BLK), lambda i, j: (0, j)),
        ],
        out_specs=pl.BlockSpec((1, 1, N_TOK_BLK), lambda i, j: (i, 0, 0)),
        out_shape=jax.ShapeDtypeStruct((nt, 1, N_TOK_BLK), jnp.int32),
        scratch_shapes=[
            pltpu.VMEM((N_TOK_BLK, 1), jnp.float32),
            pltpu.VMEM((N_TOK_BLK, 1), jnp.int32),
        ],
    )(ze2, embedW, zen, wn)

    idx_flat = idx3.reshape(N)
    table = jnp.pad(embedW, ((0, 0), (0, 128 - D)))
    zq_pad = _sc_gather(table, idx_flat, N, 128)

    zq_st, commit = pl.pallas_call(
        functools.partial(_epilogue_body, n_tok_blocks=nt,
                          inv_count=1.0 / float(N * D)),
        grid=(nt,),
        in_specs=[
            pl.BlockSpec((N_TOK_BLK, D), lambda i: (i, 0)),
            pl.BlockSpec((N_TOK_BLK, 128), lambda i: (i, 0)),
        ],
        out_specs=[
            pl.BlockSpec((N_TOK_BLK, D), lambda i: (i, 0)),
            pl.BlockSpec((1, 1), lambda i: (0, 0)),
        ],
        out_shape=[
            jax.ShapeDtypeStruct((N, D), jnp.float32),
            jax.ShapeDtypeStruct((1, 1), jnp.float32),
        ],
        scratch_shapes=[pltpu.VMEM((1, 1), jnp.float32)],
    )(ze_flat, zq_pad)

    return (zq_st.reshape(B, T, D), commit[0, 0], idx3.reshape(B, T))


# speedup vs baseline: 2.7485x; 1.0584x over previous
import functools

import jax
import jax.numpy as jnp
from jax import lax
from jax.experimental import pallas as pl
from jax.experimental.pallas import tpu as pltpu
from jax.experimental.pallas import tpu_sc as plsc

N_TOK_BLK = 2048
N_CODE_BLK = 2048


def _argmin_body(ze2_ref, w_ref, zen_ref, wn_ref, idx_ref,
                 racc_ref, ridx_ref, *, n_code_blocks):
    j = pl.program_id(1)
    tb = ze2_ref.shape[0]
    cb = w_ref.shape[0]

    mm2 = jax.lax.dot_general(ze2_ref[...], w_ref[...], (((1,), (1,)), ((), ())),
                              preferred_element_type=jnp.float32)
    dist = (zen_ref[...] + mm2) + wn_ref[...]

    bmin = jnp.min(dist, axis=1, keepdims=True)
    col = jax.lax.broadcasted_iota(jnp.int32, (tb, cb), 1)
    bidx = jnp.min(jnp.where(dist == bmin, col, 2 ** 30),
                   axis=1, keepdims=True)
    bmin_bf = bmin.astype(jnp.bfloat16).astype(jnp.float32)

    @pl.when(j == 0)
    def _():
        racc_ref[...] = bmin_bf
        ridx_ref[...] = bidx

    @pl.when(j > 0)
    def _():
        upd = bmin < racc_ref[...]
        racc_ref[...] = jnp.where(upd, bmin_bf, racc_ref[...])
        ridx_ref[...] = jnp.where(upd, bidx + j * cb, ridx_ref[...])

    @pl.when(j == n_code_blocks - 1)
    def _():
        idx_ref[...] = jnp.reshape(ridx_ref[...], (1, 1, tb))


def _epilogue_body(ze_ref, zq_ref, st_ref, commit_ref, cacc_ref,
                   *, n_tok_blocks, inv_count):
    i = pl.program_id(0)
    ze = ze_ref[...]
    zq = zq_ref[:, : ze.shape[1]]
    st_ref[...] = ze + (zq - ze)
    partial = jnp.reshape(jnp.sum((zq - ze) ** 2), (1, 1))

    @pl.when(i == 0)
    def _():
        cacc_ref[...] = partial

    @pl.when(i > 0)
    def _():
        cacc_ref[...] = cacc_ref[...] + partial

    @pl.when(i == n_tok_blocks - 1)
    def _():
        commit_ref[...] = cacc_ref[...] * inv_count


def _sc_gather(table, idx, n, d):
    info = plsc.get_sparse_core_info()
    n_workers = info.num_cores * info.num_subcores
    b_per_w = n // n_workers
    chunk = 128
    n_chunks = b_per_w // chunk
    mesh = plsc.VectorSubcoreMesh(core_axis_name="c", subcore_axis_name="s")

    @functools.partial(
        pl.kernel, mesh=mesh,
        out_type=jax.ShapeDtypeStruct((n, d), jnp.float32),
        scratch_types=[
            pltpu.VMEM((n_chunks, chunk), jnp.int32),
            pltpu.VMEM((b_per_w, d), jnp.float32),
            pltpu.SemaphoreType.DMA,
        ],
    )
    def gather_kernel(table_hbm, idx_hbm, out_hbm, idx_v, rows_v, sem):
        wid = lax.axis_index("s") * info.num_cores + lax.axis_index("c")
        base = wid * b_per_w
        copies = []
        for k in range(n_chunks):
            pltpu.sync_copy(idx_hbm.at[pl.ds(base + k * chunk, chunk)],
                            idx_v.at[k])
            copies.append(pltpu.async_copy(
                table_hbm.at[idx_v.at[k]],
                rows_v.at[pl.ds(k * chunk, chunk)], sem))
        for c in copies:
            c.wait()
        pltpu.sync_copy(rows_v, out_hbm.at[pl.ds(base, b_per_w)])

    return gather_kernel(table, idx)


def kernel(ze, embedW):
    B, T, D = ze.shape
    K = embedW.shape[0]
    N = B * T
    ze_flat = ze.reshape(-1, D)
    zen = jnp.sum(ze_flat ** 2, axis=1, keepdims=True)
    wn = jnp.sum(embedW.T ** 2, axis=0, keepdims=True)
    ze2 = -2.0 * ze_flat

    nt = N // N_TOK_BLK
    nk = K // N_CODE_BLK

    idx3 = pl.pallas_call(
        functools.partial(_argmin_body, n_code_blocks=nk),
        grid=(nt, nk),
        in_specs=[
            pl.BlockSpec((N_TOK_BLK, D), lambda i, j: (i, 0)),
            pl.BlockSpec((N_CODE_BLK, D), lambda i, j: (j, 0)),
            pl.BlockSpec((N_TOK_BLK, 1), lambda i, j: (i, 0)),
            pl.BlockSpec((1, N_CODE_BLK), lambda i, j: (0, j)),
        ],
        out_specs=pl.BlockSpec((1, 1, N_TOK_BLK), lambda i, j: (i, 0, 0)),
        out_shape=jax.ShapeDtypeStruct((nt, 1, N_TOK_BLK), jnp.int32),
        scratch_shapes=[
            pltpu.VMEM((N_TOK_BLK, 1), jnp.float32),
            pltpu.VMEM((N_TOK_BLK, 1), jnp.int32),
        ],
    )(ze2, embedW, zen, wn)

    idx_flat = idx3.reshape(N)
    table = jnp.pad(embedW, ((0, 0), (0, 128 - D)))
    zq_pad = _sc_gather(table, idx_flat, N, 128)

    zq_st, commit = pl.pallas_call(
        functools.partial(_epilogue_body, n_tok_blocks=nt,
                          inv_count=1.0 / float(N * D)),
        grid=(nt,),
        in_specs=[
            pl.BlockSpec((N_TOK_BLK, D), lambda i: (i, 0)),
            pl.BlockSpec((N_TOK_BLK, 128), lambda i: (i, 0)),
        ],
        out_specs=[
            pl.BlockSpec((N_TOK_BLK, D), lambda i: (i, 0)),
            pl.BlockSpec((1, 1), lambda i: (0, 0)),
        ],
        out_shape=[
            jax.ShapeDtypeStruct((N, D), jnp.float32),
            jax.ShapeDtypeStruct((1, 1), jnp.float32),
        ],
        scratch_shapes=[pltpu.VMEM((1, 1), jnp.float32)],
    )(ze_flat, zq_pad)

    return (zq_st.reshape(B, T, D), commit[0, 0], idx3.reshape(B, T))


# speedup vs baseline: 2.8131x; 1.0235x over previous
import functools

import jax
import jax.numpy as jnp
from jax import lax
from jax.experimental import pallas as pl
from jax.experimental.pallas import tpu as pltpu
from jax.experimental.pallas import tpu_sc as plsc

N_TOK_BLK = 4096
N_CODE_BLK = 2048


def _argmin_body(ze2_ref, w_ref, zen_ref, wn_ref, idx_ref,
                 racc_ref, ridx_ref, *, n_code_blocks):
    j = pl.program_id(1)
    tb = ze2_ref.shape[0]
    cb = w_ref.shape[0]

    mm2 = jax.lax.dot_general(ze2_ref[...], w_ref[...], (((1,), (1,)), ((), ())),
                              preferred_element_type=jnp.float32)
    dist = (zen_ref[...] + mm2) + wn_ref[...]

    bmin = jnp.min(dist, axis=1, keepdims=True)
    col = jax.lax.broadcasted_iota(jnp.int32, (tb, cb), 1)
    bidx = jnp.min(jnp.where(dist == bmin, col, 2 ** 30),
                   axis=1, keepdims=True)
    bmin_bf = bmin.astype(jnp.bfloat16).astype(jnp.float32)

    @pl.when(j == 0)
    def _():
        racc_ref[...] = bmin_bf
        ridx_ref[...] = bidx

    @pl.when(j > 0)
    def _():
        upd = bmin < racc_ref[...]
        racc_ref[...] = jnp.where(upd, bmin_bf, racc_ref[...])
        ridx_ref[...] = jnp.where(upd, bidx + j * cb, ridx_ref[...])

    @pl.when(j == n_code_blocks - 1)
    def _():
        idx_ref[...] = jnp.reshape(ridx_ref[...], (1, 1, tb))


def _epilogue_body(ze_ref, zq_ref, st_ref, commit_ref, cacc_ref,
                   *, n_tok_blocks, inv_count):
    i = pl.program_id(0)
    ze = ze_ref[...]
    zq = zq_ref[:, : ze.shape[1]]
    st_ref[...] = ze + (zq - ze)
    partial = jnp.reshape(jnp.sum((zq - ze) ** 2), (1, 1))

    @pl.when(i == 0)
    def _():
        cacc_ref[...] = partial

    @pl.when(i > 0)
    def _():
        cacc_ref[...] = cacc_ref[...] + partial

    @pl.when(i == n_tok_blocks - 1)
    def _():
        commit_ref[...] = cacc_ref[...] * inv_count


def _sc_gather(table, idx, n, d):
    info = plsc.get_sparse_core_info()
    n_workers = info.num_cores * info.num_subcores
    b_per_w = n // n_workers
    chunk = 128
    n_chunks = b_per_w // chunk
    mesh = plsc.VectorSubcoreMesh(core_axis_name="c", subcore_axis_name="s")

    @functools.partial(
        pl.kernel, mesh=mesh,
        out_type=jax.ShapeDtypeStruct((n, d), jnp.float32),
        scratch_types=[
            pltpu.VMEM((n_chunks, chunk), jnp.int32),
            pltpu.VMEM((b_per_w, d), jnp.float32),
            pltpu.SemaphoreType.DMA,
        ],
    )
    def gather_kernel(table_hbm, idx_hbm, out_hbm, idx_v, rows_v, sem):
        wid = lax.axis_index("s") * info.num_cores + lax.axis_index("c")
        base = wid * b_per_w
        copies = []
        for k in range(n_chunks):
            pltpu.sync_copy(idx_hbm.at[pl.ds(base + k * chunk, chunk)],
                            idx_v.at[k])
            copies.append(pltpu.async_copy(
                table_hbm.at[idx_v.at[k]],
                rows_v.at[pl.ds(k * chunk, chunk)], sem))
        for c in copies:
            c.wait()
        pltpu.sync_copy(rows_v, out_hbm.at[pl.ds(base, b_per_w)])

    return gather_kernel(table, idx)


def kernel(ze, embedW):
    B, T, D = ze.shape
    K = embedW.shape[0]
    N = B * T
    ze_flat = ze.reshape(-1, D)
    zen = jnp.sum(ze_flat ** 2, axis=1, keepdims=True)
    wn = jnp.sum(embedW.T ** 2, axis=0, keepdims=True)
    ze2 = -2.0 * ze_flat

    nt = N // N_TOK_BLK
    nk = K // N_CODE_BLK

    idx3 = pl.pallas_call(
        functools.partial(_argmin_body, n_code_blocks=nk),
        grid=(nt, nk),
        in_specs=[
            pl.BlockSpec((N_TOK_BLK, D), lambda i, j: (i, 0)),
            pl.BlockSpec((N_CODE_BLK, D), lambda i, j: (j, 0)),
            pl.BlockSpec((N_TOK_BLK, 1), lambda i, j: (i, 0)),
            pl.BlockSpec((1, N_CODE_BLK), lambda i, j: (0, j)),
        ],
        out_specs=pl.BlockSpec((1, 1, N_TOK_BLK), lambda i, j: (i, 0, 0)),
        out_shape=jax.ShapeDtypeStruct((nt, 1, N_TOK_BLK), jnp.int32),
        scratch_shapes=[
            pltpu.VMEM((N_TOK_BLK, 1), jnp.float32),
            pltpu.VMEM((N_TOK_BLK, 1), jnp.int32),
        ],
    )(ze2, embedW, zen, wn)

    idx_flat = idx3.reshape(N)
    table = jnp.pad(embedW, ((0, 0), (0, 128 - D)))
    zq_pad = _sc_gather(table, idx_flat, N, 128)

    zq_st, commit = pl.pallas_call(
        functools.partial(_epilogue_body, n_tok_blocks=nt,
                          inv_count=1.0 / float(N * D)),
        grid=(nt,),
        in_specs=[
            pl.BlockSpec((N_TOK_BLK, D), lambda i: (i, 0)),
            pl.BlockSpec((N_TOK_BLK, 128), lambda i: (i, 0)),
        ],
        out_specs=[
            pl.BlockSpec((N_TOK_BLK, D), lambda i: (i, 0)),
            pl.BlockSpec((1, 1), lambda i: (0, 0)),
        ],
        out_shape=[
            jax.ShapeDtypeStruct((N, D), jnp.float32),
            jax.ShapeDtypeStruct((1, 1), jnp.float32),
        ],
        scratch_shapes=[pltpu.VMEM((1, 1), jnp.float32)],
    )(ze_flat, zq_pad)

    return (zq_st.reshape(B, T, D), commit[0, 0], idx3.reshape(B, T))
